# baseline (speedup 1.0000x reference)
import functools

import jax
import jax.numpy as jnp
from jax import lax
from jax.experimental import pallas as pl
from jax.experimental.pallas import tpu as pltpu
from jax.experimental.pallas import tpu_sc as plsc

_NC, _NS = 2, 16


def _sc_gather(idx_flat, table, chunk, chunks_per_worker):
    n_rows = idx_flat.shape[0]
    d = table.shape[1]
    per_worker = chunks_per_worker * chunk
    assert n_rows == per_worker * _NC * _NS
    mesh = plsc.VectorSubcoreMesh(core_axis_name="c", subcore_axis_name="s")

    @functools.partial(
        pl.kernel,
        mesh=mesh,
        out_type=jax.ShapeDtypeStruct((n_rows, d), jnp.int32),
        scratch_types=[
            pltpu.VMEM((per_worker,), jnp.int32),
            pltpu.VMEM((chunk, d), jnp.int32),
            pltpu.SemaphoreType.DMA,
        ],
    )
    def gather_k(idx_hbm, table_hbm, out_hbm, idx_v, rows_v, sem):
        wid = lax.axis_index("s") * _NC + lax.axis_index("c")
        base = wid * per_worker
        pltpu.sync_copy(idx_hbm.at[pl.ds(base, per_worker)], idx_v)
        for j in range(chunks_per_worker):
            pltpu.async_copy(
                table_hbm.at[idx_v.at[pl.ds(j * chunk, chunk)]], rows_v, sem
            ).wait()
            pltpu.sync_copy(rows_v, out_hbm.at[pl.ds(base + j * chunk, chunk)])

    return gather_k(idx_flat, table)


def _tc_combine(hidden_states, W, b2, mt2, pos, rb):
    B, Kc, E = hidden_states.shape
    D = W.shape[1]
    K_total = pos.shape[1]
    n_ctx_blocks = Kc // rb
    n_blocks = K_total // rb

    Dp = pos.shape[2]

    def body(hs_ref, w_ref, b_ref, mt_ref, pos_ref, out_ref):
        r = pl.program_id(1)

        @pl.when(r < n_ctx_blocks)
        def _():
            acc = jax.lax.dot_general(
                hs_ref[0].astype(jnp.bfloat16), w_ref[...].astype(jnp.bfloat16),
                (((1,), (0,)), ((), ())),
                preferred_element_type=jnp.float32,
            )
            out_ref[0] = acc + b_ref[...] + pos_ref[0, :, :D].astype(jnp.float32)

        @pl.when(r >= n_ctx_blocks)
        def _():
            out_ref[0] = pos_ref[0, :, :D].astype(jnp.float32) + mt_ref[...]

    return pl.pallas_call(
        body,
        grid=(B, n_blocks),
        in_specs=[
            pl.BlockSpec((1, rb, E),
                         lambda i, r: (i, jnp.minimum(r, n_ctx_blocks - 1), 0)),
            pl.BlockSpec((E, D), lambda i, r: (0, 0)),
            pl.BlockSpec((1, D), lambda i, r: (0, 0)),
            pl.BlockSpec((1, D), lambda i, r: (0, 0)),
            pl.BlockSpec((1, rb, Dp), lambda i, r: (i, r, 0)),
        ],
        out_specs=pl.BlockSpec((1, rb, D), lambda i, r: (i, r, 0)),
        out_shape=jax.ShapeDtypeStruct((B, K_total, D), jnp.float32),
    )(hidden_states, W, b2, mt2, pos)


def kernel(hidden_states, context_mask, target_mask, mask_index, W, b, mask_token, pos_embed):
    B, Kc, E = hidden_states.shape
    Kt = target_mask.shape[1]
    D = W.shape[1]
    P = pos_embed.shape[0]
    K_total = Kc + Kt

    masks = jnp.concatenate([context_mask, target_mask], axis=1)
    table_bf = pos_embed.astype(jnp.bfloat16)
    table_i32 = jax.lax.bitcast_convert_type(
        table_bf.reshape(P, D // 2, 2), jnp.int32)
    wp = 256
    table_i32 = jnp.pad(table_i32, ((0, 0), (0, wp - D // 2)))

    chunk = 128
    n_rows = B * K_total
    chunks_per_worker = n_rows // (_NC * _NS * chunk)

    pos_i32 = _sc_gather(masks.reshape(n_rows), table_i32, chunk,
                         chunks_per_worker)
    pos_all = jax.lax.bitcast_convert_type(
        pos_i32, jnp.bfloat16).reshape(B, K_total, 2 * wp)

    b2 = b.reshape(1, D)
    mt2 = mask_token.reshape(1, D)
    rb = 1152
    embeddings = _tc_combine(hidden_states, W, b2, mt2, pos_all, rb)
    return (embeddings, masks)


# speedup vs baseline: 4.1282x; 4.1282x over previous
import functools

import jax
import jax.numpy as jnp
from jax import lax
from jax.experimental import pallas as pl
from jax.experimental.pallas import tpu as pltpu
from jax.experimental.pallas import tpu_sc as plsc

_NC, _NS = 2, 16


def _sc_gather(idx_flat, table, chunk, chunks_per_worker):
    n_rows = idx_flat.shape[0]
    d = table.shape[1]
    per_worker = chunks_per_worker * chunk
    assert n_rows == per_worker * _NC * _NS
    mesh = plsc.VectorSubcoreMesh(core_axis_name="c", subcore_axis_name="s")

    @functools.partial(
        pl.kernel,
        mesh=mesh,
        out_type=jax.ShapeDtypeStruct((n_rows, d), jnp.int32),
        scratch_types=[
            pltpu.VMEM((per_worker,), jnp.int32),
            pltpu.VMEM((chunk, d), jnp.int32),
            pltpu.SemaphoreType.DMA,
        ],
    )
    def gather_k(idx_hbm, table_hbm, out_hbm, idx_v, rows_v, sem):
        wid = lax.axis_index("s") * _NC + lax.axis_index("c")
        base = wid * per_worker
        pltpu.sync_copy(idx_hbm.at[pl.ds(base, per_worker)], idx_v)
        for j in range(chunks_per_worker):
            pltpu.async_copy(
                table_hbm.at[idx_v.at[pl.ds(j * chunk, chunk)]], rows_v, sem
            ).wait()
            pltpu.sync_copy(rows_v, out_hbm.at[pl.ds(base + j * chunk, chunk)])

    return gather_k(idx_flat, table)


def _tc_combine(hidden_states, W, b2, mt2, pos, rb):
    B, Kc, E = hidden_states.shape
    D = W.shape[1]
    K_total = pos.shape[1]
    n_ctx_blocks = Kc // rb
    n_blocks = K_total // rb

    wp = pos.shape[2]

    def unpack(pw):
        lo = jax.lax.bitcast_convert_type(pw << 16, jnp.float32)
        hi = jax.lax.bitcast_convert_type(pw & jnp.int32(-65536), jnp.float32)
        return lo, hi

    def body(hs_ref, w_ref, b_ref, mt_ref, pos_ref, out_ref):
        r = pl.program_id(1)
        lo, hi = unpack(pos_ref[0])

        @pl.when(r < n_ctx_blocks)
        def _():
            acc = jax.lax.dot_general(
                hs_ref[0].astype(jnp.bfloat16), w_ref[...].astype(jnp.bfloat16),
                (((1,), (0,)), ((), ())),
                preferred_element_type=jnp.float32,
            ) + b_ref[...]
            out_ref[0, :, :wp] = acc[:, :wp] + lo
            out_ref[0, :, wp:] = acc[:, wp:] + hi[:, :D - wp]

        @pl.when(r >= n_ctx_blocks)
        def _():
            out_ref[0, :, :wp] = lo + mt_ref[0, :wp]
            out_ref[0, :, wp:] = hi[:, :D - wp] + mt_ref[0, wp:]

    return pl.pallas_call(
        body,
        grid=(B, n_blocks),
        in_specs=[
            pl.BlockSpec((1, rb, E),
                         lambda i, r: (i, jnp.minimum(r, n_ctx_blocks - 1), 0)),
            pl.BlockSpec((E, D), lambda i, r: (0, 0)),
            pl.BlockSpec((1, D), lambda i, r: (0, 0)),
            pl.BlockSpec((1, D), lambda i, r: (0, 0)),
            pl.BlockSpec((1, rb, wp), lambda i, r: (i, r, 0)),
        ],
        out_specs=pl.BlockSpec((1, rb, D), lambda i, r: (i, r, 0)),
        out_shape=jax.ShapeDtypeStruct((B, K_total, D), jnp.float32),
    )(hidden_states, W, b2, mt2, pos)


def kernel(hidden_states, context_mask, target_mask, mask_index, W, b, mask_token, pos_embed):
    B, Kc, E = hidden_states.shape
    Kt = target_mask.shape[1]
    D = W.shape[1]
    P = pos_embed.shape[0]
    K_total = Kc + Kt

    masks = jnp.concatenate([context_mask, target_mask], axis=1)
    wp = 256
    pos_bf = pos_embed.astype(jnp.bfloat16)
    pos_pad = jnp.pad(pos_bf, ((0, 0), (0, 2 * wp - D)))
    u32 = jax.lax.bitcast_convert_type(pos_pad, jnp.uint16).astype(jnp.uint32)
    table_i32 = jax.lax.bitcast_convert_type(
        (u32[:, wp:] << 16) | u32[:, :wp], jnp.int32)

    chunk = 128
    n_rows = B * K_total
    chunks_per_worker = n_rows // (_NC * _NS * chunk)

    pos_all = _sc_gather(masks.reshape(n_rows), table_i32, chunk,
                         chunks_per_worker).reshape(B, K_total, wp)

    b2 = b.reshape(1, D)
    mt2 = mask_token.reshape(1, D)
    rb = 1152
    embeddings = _tc_combine(hidden_states, W, b2, mt2, pos_all, rb)
    return (embeddings, masks)


# speedup vs baseline: 4.1440x; 1.0038x over previous
import functools

import jax
import jax.numpy as jnp
from jax import lax
from jax.experimental import pallas as pl
from jax.experimental.pallas import tpu as pltpu
from jax.experimental.pallas import tpu_sc as plsc

_NC, _NS = 2, 16


def _sc_gather(idx_flat, table, chunk, chunks_per_worker):
    n_rows = idx_flat.shape[0]
    d = table.shape[1]
    per_worker = chunks_per_worker * chunk
    assert n_rows == per_worker * _NC * _NS
    mesh = plsc.VectorSubcoreMesh(core_axis_name="c", subcore_axis_name="s")

    @functools.partial(
        pl.kernel,
        mesh=mesh,
        out_type=jax.ShapeDtypeStruct((n_rows, d), jnp.int32),
        scratch_types=[
            pltpu.VMEM((per_worker,), jnp.int32),
            pltpu.VMEM((chunk, d), jnp.int32),
            pltpu.SemaphoreType.DMA,
        ],
    )
    def gather_k(idx_hbm, table_hbm, out_hbm, idx_v, rows_v, sem):
        wid = lax.axis_index("s") * _NC + lax.axis_index("c")
        base = wid * per_worker
        pltpu.sync_copy(idx_hbm.at[pl.ds(base, per_worker)], idx_v)
        for j in range(chunks_per_worker):
            pltpu.async_copy(
                table_hbm.at[idx_v.at[pl.ds(j * chunk, chunk)]], rows_v, sem
            ).wait()
            pltpu.sync_copy(rows_v, out_hbm.at[pl.ds(base + j * chunk, chunk)])

    return gather_k(idx_flat, table)


def _tc_combine(hidden_states, W, b2, mt2, pos, rb):
    B, Kc, E = hidden_states.shape
    D = W.shape[1]
    K_total = pos.shape[1]
    n_ctx_blocks = Kc // rb
    n_blocks = K_total // rb

    wp = pos.shape[2]

    def unpack(pw):
        lo = jax.lax.bitcast_convert_type(pw << 16, jnp.float32)
        hi = jax.lax.bitcast_convert_type(pw & jnp.int32(-65536), jnp.float32)
        return jnp.concatenate([lo, hi[:, :D - wp]], axis=1)

    def body(hs_ref, w_ref, b_ref, mt_ref, pos_ref, out_ref):
        r = pl.program_id(1)
        pos = unpack(pos_ref[0])

        @pl.when(r < n_ctx_blocks)
        def _():
            acc = jax.lax.dot_general(
                hs_ref[0].astype(jnp.bfloat16), w_ref[...].astype(jnp.bfloat16),
                (((1,), (0,)), ((), ())),
                preferred_element_type=jnp.float32,
            )
            out_ref[0] = acc + b_ref[...] + pos

        @pl.when(r >= n_ctx_blocks)
        def _():
            out_ref[0] = pos + mt_ref[...]

    return pl.pallas_call(
        body,
        grid=(B, n_blocks),
        in_specs=[
            pl.BlockSpec((1, rb, E),
                         lambda i, r: (i, jnp.minimum(r, n_ctx_blocks - 1), 0)),
            pl.BlockSpec((E, D), lambda i, r: (0, 0)),
            pl.BlockSpec((1, D), lambda i, r: (0, 0)),
            pl.BlockSpec((1, D), lambda i, r: (0, 0)),
            pl.BlockSpec((1, rb, wp), lambda i, r: (i, r, 0)),
        ],
        out_specs=pl.BlockSpec((1, rb, D), lambda i, r: (i, r, 0)),
        out_shape=jax.ShapeDtypeStruct((B, K_total, D), jnp.float32),
    )(hidden_states, W, b2, mt2, pos)


def kernel(hidden_states, context_mask, target_mask, mask_index, W, b, mask_token, pos_embed):
    B, Kc, E = hidden_states.shape
    Kt = target_mask.shape[1]
    D = W.shape[1]
    P = pos_embed.shape[0]
    K_total = Kc + Kt

    masks = jnp.concatenate([context_mask, target_mask], axis=1)
    wp = 256
    pos_bf = pos_embed.astype(jnp.bfloat16)
    pos_pad = jnp.pad(pos_bf, ((0, 0), (0, 2 * wp - D)))
    u32 = jax.lax.bitcast_convert_type(pos_pad, jnp.uint16).astype(jnp.uint32)
    table_i32 = jax.lax.bitcast_convert_type(
        (u32[:, wp:] << 16) | u32[:, :wp], jnp.int32)

    chunk = 128
    n_rows = B * K_total
    chunks_per_worker = n_rows // (_NC * _NS * chunk)

    pos_all = _sc_gather(masks.reshape(n_rows), table_i32, chunk,
                         chunks_per_worker).reshape(B, K_total, wp)

    b2 = b.reshape(1, D)
    mt2 = mask_token.reshape(1, D)
    rb = 1152
    embeddings = _tc_combine(hidden_states, W, b2, mt2, pos_all, rb)
    return (embeddings, masks)


# speedup vs baseline: 4.3179x; 1.0420x over previous
import functools

import jax
import jax.numpy as jnp
from jax import lax
from jax.experimental import pallas as pl
from jax.experimental.pallas import tpu as pltpu
from jax.experimental.pallas import tpu_sc as plsc

_NC, _NS = 2, 16


def _sc_gather(idx_flat, table_pk, table_t, kc, kt, d, wp, chunk,
               chunks_per_worker):
    n_rows = idx_flat.shape[0]
    per_worker = chunks_per_worker * chunk
    assert n_rows == per_worker * _NC * _NS
    rpb = (kc + kt) // per_worker
    ctx_rpb = kc // per_worker
    n_ctx = (n_rows // (kc + kt)) * kc
    mesh = plsc.VectorSubcoreMesh(core_axis_name="c", subcore_axis_name="s")

    @functools.partial(
        pl.kernel,
        mesh=mesh,
        out_type=[
            jax.ShapeDtypeStruct((n_ctx, wp), jnp.int32),
            jax.ShapeDtypeStruct((n_rows, d), jnp.float32),
        ],
        scratch_types=[
            pltpu.VMEM((per_worker,), jnp.int32),
            pltpu.VMEM((chunk, wp), jnp.int32),
            pltpu.VMEM((chunk, d), jnp.float32),
            pltpu.SemaphoreType.DMA,
        ],
    )
    def gather_k(idx_hbm, tpk_hbm, tt_hbm, ctx_hbm, full_hbm,
                 idx_v, rows_i, rows_f, sem):
        wid = lax.axis_index("s") * _NC + lax.axis_index("c")
        base = wid * per_worker
        pltpu.sync_copy(idx_hbm.at[pl.ds(base, per_worker)], idx_v)
        rgn = lax.rem(wid, rpb)
        is_ctx = rgn < ctx_rpb

        @pl.when(is_ctx)
        def _():
            cbase = (wid // rpb) * kc + rgn * per_worker
            for j in range(chunks_per_worker):
                pltpu.async_copy(
                    tpk_hbm.at[idx_v.at[pl.ds(j * chunk, chunk)]], rows_i, sem
                ).wait()
                pltpu.sync_copy(rows_i, ctx_hbm.at[pl.ds(cbase + j * chunk, chunk)])

        @pl.when(jnp.logical_not(is_ctx))
        def _():
            for j in range(chunks_per_worker):
                pltpu.async_copy(
                    tt_hbm.at[idx_v.at[pl.ds(j * chunk, chunk)]], rows_f, sem
                ).wait()
                pltpu.sync_copy(rows_f, full_hbm.at[pl.ds(base + j * chunk, chunk)])

    return gather_k(idx_flat, table_pk, table_t)


def _tc_combine(hidden_states, W, b2, pos_pk, x, rb):
    B, Kc, E = hidden_states.shape
    D = W.shape[1]
    K_total = x.shape[1]
    wp = pos_pk.shape[2]
    n_ctx_blocks = Kc // rb

    def body(hs_ref, w_ref, b_ref, pos_ref, x_ref, out_ref):
        pw = pos_ref[0]
        lo = jax.lax.bitcast_convert_type(pw << 16, jnp.float32)
        hi = jax.lax.bitcast_convert_type(pw & jnp.int32(-65536), jnp.float32)
        pos = jnp.concatenate([lo, hi[:, :D - wp]], axis=1)
        acc = jax.lax.dot_general(
            hs_ref[0].astype(jnp.bfloat16), w_ref[...].astype(jnp.bfloat16),
            (((1,), (0,)), ((), ())),
            preferred_element_type=jnp.float32,
        )
        out_ref[0] = acc + b_ref[...] + pos

    return pl.pallas_call(
        body,
        grid=(B, n_ctx_blocks),
        in_specs=[
            pl.BlockSpec((1, rb, E), lambda i, r: (i, r, 0)),
            pl.BlockSpec((E, D), lambda i, r: (0, 0)),
            pl.BlockSpec((1, D), lambda i, r: (0, 0)),
            pl.BlockSpec((1, rb, wp), lambda i, r: (i, r, 0)),
            pl.BlockSpec(memory_space=pl.ANY),
        ],
        out_specs=pl.BlockSpec((1, rb, D), lambda i, r: (i, r, 0)),
        out_shape=jax.ShapeDtypeStruct((B, K_total, D), jnp.float32),
        input_output_aliases={4: 0},
    )(hidden_states, W, b2, pos_pk, x)


def kernel(hidden_states, context_mask, target_mask, mask_index, W, b, mask_token, pos_embed):
    B, Kc, E = hidden_states.shape
    Kt = target_mask.shape[1]
    D = W.shape[1]
    P = pos_embed.shape[0]
    K_total = Kc + Kt

    masks = jnp.concatenate([context_mask, target_mask], axis=1)

    wp = 256
    xb = jax.lax.bitcast_convert_type(pos_embed, jnp.uint32)
    r16 = (xb + jnp.uint32(0x7FFF) + ((xb >> 16) & jnp.uint32(1))) >> 16
    hi = jnp.pad(r16[:, wp:], ((0, 0), (0, 2 * wp - D)))
    table_pk = jax.lax.bitcast_convert_type(r16[:, :wp] | (hi << 16), jnp.int32)
    table_t = pos_embed + mask_token[0]

    chunk = 128
    n_rows = B * K_total
    chunks_per_worker = n_rows // (_NC * _NS * chunk)

    pos_pk, x = _sc_gather(masks.reshape(n_rows), table_pk, table_t,
                           Kc, Kt, D, wp, chunk, chunks_per_worker)
    pos_pk = pos_pk.reshape(B, Kc, wp)
    x = x.reshape(B, K_total, D)

    b2 = b.reshape(1, D)
    rb = 1728
    embeddings = _tc_combine(hidden_states, W, b2, pos_pk, x, rb)
    return (embeddings, masks)


# speedup vs baseline: 4.3921x; 1.0172x over previous
import functools

import jax
import jax.numpy as jnp
from jax import lax
from jax.experimental import pallas as pl
from jax.experimental.pallas import tpu as pltpu
from jax.experimental.pallas import tpu_sc as plsc

_NC, _NS = 2, 16


def _sc_gather(idx_flat, table_pk, table_t, kc, kt, d, wp, chunk,
               chunks_per_worker):
    n_rows = idx_flat.shape[0]
    per_worker = chunks_per_worker * chunk
    assert n_rows == per_worker * _NC * _NS
    rpb = (kc + kt) // per_worker
    ctx_rpb = kc // per_worker
    n_ctx = (n_rows // (kc + kt)) * kc
    mesh = plsc.VectorSubcoreMesh(core_axis_name="c", subcore_axis_name="s")

    @functools.partial(
        pl.kernel,
        mesh=mesh,
        out_type=[
            jax.ShapeDtypeStruct((n_ctx, wp), jnp.int32),
            jax.ShapeDtypeStruct((n_rows, d), jnp.float32),
        ],
        scratch_types=[
            pltpu.VMEM((per_worker,), jnp.int32),
            pltpu.VMEM((chunk, wp), jnp.int32),
            pltpu.VMEM((chunk, d), jnp.float32),
            pltpu.SemaphoreType.DMA,
        ],
    )
    def gather_k(idx_hbm, tpk_hbm, tt_hbm, ctx_hbm, full_hbm,
                 idx_v, rows_i, rows_f, sem):
        wid = lax.axis_index("s") * _NC + lax.axis_index("c")
        base = wid * per_worker
        pltpu.sync_copy(idx_hbm.at[pl.ds(base, per_worker)], idx_v)
        rgn = lax.rem(wid, rpb)
        is_ctx = rgn < ctx_rpb

        @pl.when(is_ctx)
        def _():
            cbase = (wid // rpb) * kc + rgn * per_worker
            for j in range(chunks_per_worker):
                pltpu.async_copy(
                    tpk_hbm.at[idx_v.at[pl.ds(j * chunk, chunk)]], rows_i, sem
                ).wait()
                pltpu.sync_copy(rows_i, ctx_hbm.at[pl.ds(cbase + j * chunk, chunk)])

        @pl.when(jnp.logical_not(is_ctx))
        def _():
            for j in range(chunks_per_worker):
                pltpu.async_copy(
                    tt_hbm.at[idx_v.at[pl.ds(j * chunk, chunk)]], rows_f, sem
                ).wait()
                pltpu.sync_copy(rows_f, full_hbm.at[pl.ds(base + j * chunk, chunk)])

    return gather_k(idx_flat, table_pk, table_t)


def _tc_combine(hidden_states, W, b2, pos_pk, x, rb):
    B, Kc, E = hidden_states.shape
    D = W.shape[1]
    K_total = x.shape[1]
    wp = pos_pk.shape[2]
    n_ctx_blocks = Kc // rb

    def body(hs_ref, w_ref, b_ref, pos_ref, x_ref, out_ref):
        pw = pos_ref[0]
        lo = jax.lax.bitcast_convert_type(pw << 16, jnp.float32)
        hi = jax.lax.bitcast_convert_type(pw & jnp.int32(-65536), jnp.float32)
        pos = jnp.concatenate([lo, hi[:, :D - wp]], axis=1)
        acc = jax.lax.dot_general(
            hs_ref[0].astype(jnp.bfloat16), w_ref[...].astype(jnp.bfloat16),
            (((1,), (0,)), ((), ())),
            preferred_element_type=jnp.float32,
        )
        out_ref[0] = acc + b_ref[...] + pos

    return pl.pallas_call(
        body,
        grid=(B, n_ctx_blocks),
        in_specs=[
            pl.BlockSpec((1, rb, E), lambda i, r: (i, r, 0)),
            pl.BlockSpec((E, D), lambda i, r: (0, 0)),
            pl.BlockSpec((1, D), lambda i, r: (0, 0)),
            pl.BlockSpec((1, rb, wp), lambda i, r: (i, r, 0)),
            pl.BlockSpec(memory_space=pl.ANY),
        ],
        out_specs=pl.BlockSpec((1, rb, D), lambda i, r: (i, r, 0)),
        out_shape=jax.ShapeDtypeStruct((B, K_total, D), jnp.float32),
        input_output_aliases={4: 0},
    )(hidden_states, W, b2, pos_pk, x)


def kernel(hidden_states, context_mask, target_mask, mask_index, W, b, mask_token, pos_embed):
    B, Kc, E = hidden_states.shape
    Kt = target_mask.shape[1]
    D = W.shape[1]
    P = pos_embed.shape[0]
    K_total = Kc + Kt

    masks = jnp.concatenate([context_mask, target_mask], axis=1)

    wp = 256
    xb = jax.lax.bitcast_convert_type(pos_embed, jnp.uint32)
    r16 = (xb + jnp.uint32(0x7FFF) + ((xb >> 16) & jnp.uint32(1))) >> 16
    hi = jnp.pad(r16[:, wp:], ((0, 0), (0, 2 * wp - D)))
    table_pk = jax.lax.bitcast_convert_type(r16[:, :wp] | (hi << 16), jnp.int32)
    table_t = pos_embed + mask_token[0]

    chunk = 128
    n_rows = B * K_total
    chunks_per_worker = n_rows // (_NC * _NS * chunk)

    pos_pk, x = _sc_gather(masks.reshape(n_rows), table_pk, table_t,
                           Kc, Kt, D, wp, chunk, chunks_per_worker)
    pos_pk = pos_pk.reshape(B, Kc, wp)
    x = x.reshape(B, K_total, D)

    b2 = b.reshape(1, D)
    rb = 3456
    embeddings = _tc_combine(hidden_states, W, b2, pos_pk, x, rb)
    return (embeddings, masks)
